# baseline (speedup 1.0000x reference)
import functools

import jax
import jax.numpy as jnp
from jax import lax
from jax.experimental import pallas as pl
from jax.experimental.pallas import tpu as pltpu
from jax.experimental.pallas import tpu_sc as plsc

N = 100000
D = 128
B = 128
MARGIN = 0.0
EPS = 1e-06

NC = 2
NS = 16
NW = NC * NS
CHM = 128
NKM = 24
CHT = 56
SPAN = NKM * CHM + CHT
IDXROW = 3200
DUMMY = 3 * B
ACCR = 3 * B + 8


def _sc_pool_body(idx_hbm, a_hbm, p_hbm, n_hbm, out_hbm,
                  acc, buf, idxb, sem0, sem1):
    cid = lax.axis_index("c")
    sid = lax.axis_index("s")
    wid = cid * NS + sid
    s0 = 3120 * wid + 8 * jnp.maximum(0, wid - 12)
    sems = (sem0, sem1)

    def _z(i, _):
        acc[i // 8, pl.ds((i % 8) * 16, 16)] = jnp.zeros((16,), jnp.float32)
        return 0
    lax.fori_loop(0, ACCR * 8, _z, 0)

    pltpu.sync_copy(idx_hbm.at[wid], idxb)

    def _rows16(t, slot, base, nrows):
        iv = idxb[t, pl.ds(base, 16)]
        for r in range(nrows):
            seg = iv[r]
            for j in range(8):
                v = buf[slot, base % CHM + r, pl.ds(16 * j, 16)]
                plsc.addupdate(acc.at[seg, pl.ds(16 * j, 16)], v)

    def _chunk16(t, slot, k, g2):
        base = k * CHM + g2 * 16
        iv = idxb[t, pl.ds(base, 16)]
        for r in range(16):
            seg = iv[r]
            for j in range(8):
                v = buf[slot, g2 * 16 + r, pl.ds(16 * j, 16)]
                plsc.addupdate(acc.at[seg, pl.ds(16 * j, 16)], v)

    xs = (a_hbm, p_hbm, n_hbm)
    for t in range(3):
        x = xs[t]

        def _desc(k, slot, sz=CHM):
            return pltpu.make_async_copy(
                x.at[pl.ds(s0 + CHM * k, sz), :],
                buf.at[slot, pl.ds(0, sz), :], sems[slot])

        _desc(0, 0).start()
        _desc(1, 1).start()

        def _pair(q, _):
            for b in range(2):
                k = 2 * q + b
                _desc(k, b).wait()

                def _grp(g2, __):
                    _chunk16(t, b, k, g2)
                    return 0
                lax.fori_loop(0, CHM // 16, _grp, 0)

                @pl.when(k + 2 < NKM)
                def _():
                    _desc(k + 2, b).start()
            return 0
        lax.fori_loop(0, NKM // 2, _pair, 0)

        pltpu.sync_copy(x.at[pl.ds(s0 + CHM * NKM, CHT), :],
                        buf.at[0, pl.ds(0, CHT), :])

        def _tgrp(g2, _):
            _chunk16(t, 0, NKM, g2)
            return 0
        lax.fori_loop(0, 3, _tgrp, 0)
        _rows16(t, 0, NKM * CHM + 48, 8)

    pltpu.sync_copy(acc.at[pl.ds(0, 3 * B), :], out_hbm.at[wid])


_sc_pool = functools.partial(
    pl.kernel,
    out_type=jax.ShapeDtypeStruct((NW, 3 * B, D), jnp.float32),
    mesh=plsc.VectorSubcoreMesh(core_axis_name="c", subcore_axis_name="s"),
    scratch_types=[
        pltpu.VMEM((ACCR, D), jnp.float32),
        pltpu.VMEM((2, CHM, D), jnp.float32),
        pltpu.VMEM((3, IDXROW), jnp.int32),
        pltpu.SemaphoreType.DMA,
        pltpu.SemaphoreType.DMA,
    ],
)(_sc_pool_body)


def _loss_body(part_ref, agt_ref, pgt_ref, ngt_ref, out_ref):
    pooled = jnp.sum(part_ref[...], axis=0)
    a_p = pooled[0:B, :]
    p_p = pooled[B:2 * B, :]
    n_p = pooled[2 * B:3 * B, :]
    pos_d = jnp.sqrt(jnp.sum((p_p - a_p) ** 2, axis=1, keepdims=True))
    neg_d = jnp.sqrt(jnp.sum((n_p - a_p) ** 2, axis=1, keepdims=True))
    agt = agt_ref[...]
    coeff = jnp.abs(ngt_ref[...] - agt) / (jnp.abs(pgt_ref[...] - agt) + EPS)
    loss = jnp.maximum(pos_d - coeff * neg_d + MARGIN, 0.0)
    out_ref[...] = (jnp.sum(loss) / B).reshape(1, 1)


_loss = pl.pallas_call(
    _loss_body,
    out_shape=jax.ShapeDtypeStruct((1, 1), jnp.float32),
)


def _prep_idx(ab, pb, nb):
    dummy8 = jnp.full((12, 8), DUMMY, jnp.int32)
    pad = jnp.full((NW, IDXROW - SPAN), DUMMY, jnp.int32)
    rows = []
    for t, b in enumerate((ab, pb, nb)):
        arr = b.astype(jnp.int32) + t * B
        lo = arr[:12 * 3120].reshape(12, 3120)
        hi = arr[12 * 3120:].reshape(20, 3128)
        lo = jnp.concatenate([lo, dummy8], axis=1)
        full = jnp.concatenate([lo, hi])
        rows.append(jnp.concatenate([full, pad], axis=1))
    return jnp.stack(rows, axis=1)


def kernel(anchor_batch, negative_batch, positive_batch, anchor, negative,
           positive, anchor_gt, negative_gt, positive_gt):
    idx = _prep_idx(anchor_batch, positive_batch, negative_batch)
    parts = _sc_pool(idx, anchor, positive, negative)
    out = _loss(parts,
                anchor_gt.reshape(B, 1),
                positive_gt.reshape(B, 1),
                negative_gt.reshape(B, 1))
    return out[0, 0]


# speedup vs baseline: 2.0104x; 2.0104x over previous
import functools

import jax
import jax.numpy as jnp
from jax import lax
from jax.experimental import pallas as pl
from jax.experimental.pallas import tpu as pltpu
from jax.experimental.pallas import tpu_sc as plsc

N = 100000
D = 128
B = 128
MARGIN = 0.0
EPS = 1e-06

NC = 2
NS = 16
NW = NC * NS
CHM = 128
NKM = 24
CHT = 56
SPAN = NKM * CHM + CHT
DUMMY = 3 * B


def _sc_pool_body(im_hbm, it_hbm, a_hbm, p_hbm, n_hbm, out_hbm,
                  acc_sh, buf, idxm, idxt, zbuf, sem0, sem1, ssem0, ssem1):
    cid = lax.axis_index("c")
    sid = lax.axis_index("s")
    wid = cid * NS + sid
    s0 = 3120 * wid + 8 * jnp.maximum(0, wid - 12)
    sems = (sem0, sem1)
    ssems = (ssem0, ssem1)

    def _z(i, _):
        zbuf[i // 8, pl.ds((i % 8) * 16, 16)] = jnp.zeros((16,), jnp.float32)
        return 0
    lax.fori_loop(0, B * 8, _z, 0)

    @pl.when(sid == 0)
    def _():
        for t in range(3):
            pltpu.sync_copy(zbuf, acc_sh.at[pl.ds(t * B, B), :])
        pltpu.sync_copy(zbuf.at[pl.ds(0, 8), :],
                        acc_sh.at[pl.ds(3 * B, 8), :])

    plsc.subcore_barrier()

    pltpu.sync_copy(im_hbm.at[wid], idxm)
    pltpu.sync_copy(it_hbm.at[wid], idxt)

    xs = (a_hbm, p_hbm, n_hbm)
    steps = [(t, k) for t in range(3) for k in range(NKM + 1)]

    def _start(c, pb):
        t, k = steps[c]
        sz = CHM if k < NKM else CHT
        row0 = s0 + CHM * k
        return pltpu.async_copy(xs[t].at[pl.ds(row0, sz), :],
                                buf.at[pb, pl.ds(0, sz), :], sems[pb])

    def _scat(c, pb):
        t, k = steps[c]
        if k < NKM:
            return pltpu.async_copy(buf.at[pb], acc_sh.at[idxm.at[t, k]],
                                    ssems[pb], add=True)
        return pltpu.async_copy(buf.at[pb, pl.ds(0, CHT), :],
                                acc_sh.at[idxt.at[t]], ssems[pb], add=True)

    nsteps = len(steps)
    copies = [None] * nsteps
    scats = [None] * nsteps
    copies[0] = _start(0, 0)
    for c in range(nsteps):
        pb = c % 2
        if c + 1 < nsteps:
            if c >= 1:
                scats[c - 1].wait()
            copies[c + 1] = _start(c + 1, (c + 1) % 2)
        copies[c].wait()
        scats[c] = _scat(c, pb)
    scats[nsteps - 2].wait()
    scats[nsteps - 1].wait()

    plsc.subcore_barrier()

    @pl.when(sid == 0)
    def _():
        pltpu.sync_copy(acc_sh.at[pl.ds(0, 3 * B), :], out_hbm.at[cid])


_sc_pool = functools.partial(
    pl.kernel,
    out_type=jax.ShapeDtypeStruct((NC, 3 * B, D), jnp.float32),
    mesh=plsc.VectorSubcoreMesh(core_axis_name="c", subcore_axis_name="s"),
    scratch_types=[
        pltpu.VMEM_SHARED((3 * B + 8, D), jnp.float32),
        pltpu.VMEM((2, CHM, D), jnp.float32),
        pltpu.VMEM((3, NKM, CHM), jnp.int32),
        pltpu.VMEM((8, CHT), jnp.int32),
        pltpu.VMEM((B, D), jnp.float32),
        pltpu.SemaphoreType.DMA,
        pltpu.SemaphoreType.DMA,
        pltpu.SemaphoreType.DMA,
        pltpu.SemaphoreType.DMA,
    ],
)(_sc_pool_body)


def _loss_body(part_ref, agt_ref, pgt_ref, ngt_ref, out_ref):
    pooled = part_ref[0] + part_ref[1]
    a_p = pooled[0:B, :]
    p_p = pooled[B:2 * B, :]
    n_p = pooled[2 * B:3 * B, :]
    pos_d = jnp.sqrt(jnp.sum((p_p - a_p) ** 2, axis=1, keepdims=True))
    neg_d = jnp.sqrt(jnp.sum((n_p - a_p) ** 2, axis=1, keepdims=True))
    agt = agt_ref[...]
    coeff = jnp.abs(ngt_ref[...] - agt) / (jnp.abs(pgt_ref[...] - agt) + EPS)
    loss = jnp.maximum(pos_d - coeff * neg_d + MARGIN, 0.0)
    out_ref[...] = (jnp.sum(loss) / B).reshape(1, 1)


_loss = pl.pallas_call(
    _loss_body,
    out_shape=jax.ShapeDtypeStruct((1, 1), jnp.float32),
)


def _prep_idx(ab, pb, nb):
    mains, tails = [], []
    dummy8 = jnp.full((12, 8), DUMMY, jnp.int32)
    for t, b in enumerate((ab, pb, nb)):
        arr = b.astype(jnp.int32) + t * B
        lo = arr[:12 * 3120].reshape(12, 3120)
        hi = arr[12 * 3120:].reshape(20, 3128)
        main = jnp.concatenate([lo[:, :NKM * CHM], hi[:, :NKM * CHM]])
        mains.append(main.reshape(NW, NKM, CHM))
        tail_lo = jnp.concatenate([lo[:, NKM * CHM:], dummy8], axis=1)
        tails.append(jnp.concatenate([tail_lo, hi[:, NKM * CHM:]]))
    idx_main = jnp.stack(mains, axis=1)
    tail = jnp.stack(tails, axis=1)
    pad = jnp.full((NW, 5, CHT), DUMMY, jnp.int32)
    idx_tail = jnp.concatenate([tail, pad], axis=1)
    return idx_main, idx_tail


def kernel(anchor_batch, negative_batch, positive_batch, anchor, negative,
           positive, anchor_gt, negative_gt, positive_gt):
    idx_main, idx_tail = _prep_idx(anchor_batch, positive_batch,
                                   negative_batch)
    parts = _sc_pool(idx_main, idx_tail, anchor, positive, negative)
    out = _loss(parts,
                anchor_gt.reshape(B, 1),
                positive_gt.reshape(B, 1),
                negative_gt.reshape(B, 1))
    return out[0, 0]
